# baseline (speedup 1.0000x reference)
import functools
import jax
import jax.numpy as jnp
from jax import lax
from jax.experimental import pallas as pl
from jax.experimental.pallas import tpu as pltpu
from jax.experimental.pallas import tpu_sc as plsc

NUM_TOKENS = 16384 * 50
NEMB = 1000000
DIM = 64
POS = 50
SEQ = 16384
NC, NS = 2, 16
NW = NC * NS
CHUNK = 128
UNITS = NUM_TOKENS // CHUNK
UPW = UNITS // NW
SBLK = SEQ // CHUNK
NGB = 4
NTB = 2
OUTER = UPW // NGB


@functools.partial(
    pl.kernel,
    mesh=plsc.VectorSubcoreMesh(core_axis_name="c", subcore_axis_name="s"),
    out_type=jax.ShapeDtypeStruct((POS, DIM, SEQ), jnp.float32),
    scratch_types=(
        [pltpu.VMEM((UPW, CHUNK), jnp.int32)]
        + [pltpu.VMEM((CHUNK, 128), jnp.float32) for _ in range(NGB)]
        + [pltpu.VMEM((DIM, CHUNK), jnp.float32) for _ in range(NTB)]
        + [pltpu.SemaphoreType.DMA for _ in range(NGB + NTB)]
    ),
    compiler_params=pltpu.CompilerParams(use_tc_tiling_on_sc=True,
                                         needs_layout_passes=False),
)
def _embedding_gather(table_hbm, idx_hbm, out_hbm, idx_v, *scratch):
    gbuf = scratch[:NGB]
    tbuf = scratch[NGB:NGB + NTB]
    gsem = scratch[NGB + NTB:2 * NGB + NTB]
    osem = scratch[2 * NGB + NTB:]

    wid = lax.axis_index("s") * NC + lax.axis_index("c")
    ubase = wid * UPW

    pltpu.sync_copy(idx_hbm.at[wid], idx_v)

    lane = lax.iota(jnp.int32, 16)

    def fire_gather(jloc, b):
        pltpu.async_copy(table_hbm.at[idx_v.at[jloc]], gbuf[b], gsem[b])

    def gather_wait(b):
        pltpu.make_async_copy(table_hbm.at[idx_v.at[0]], gbuf[b], gsem[b]).wait()

    def transpose_unit(b, tb):
        rows = [jg * 16 + lane for jg in range(CHUNK // 16)]

        def dbody(db, carry):
            base = db * 16
            for s in range(16):
                dvec = ((lane + s) & 15) + base
                for jg in range(CHUNK // 16):
                    v = plsc.load_gather(gbuf[b], [rows[jg], dvec])
                    plsc.store_scatter(tbuf[tb], [dvec, rows[jg]], v)
            return carry

        lax.fori_loop(0, DIM // 16, dbody, 0)

    def out_start(u, tb):
        p = u // SBLK
        sb = u % SBLK
        pltpu.async_copy(tbuf[tb], out_hbm.at[p, :, pl.ds(sb * CHUNK, CHUNK)],
                         osem[tb])

    def out_wait(tb):
        pltpu.make_async_copy(tbuf[tb], out_hbm.at[0, :, pl.ds(0, CHUNK)],
                              osem[tb]).wait()

    for b in range(NGB):
        fire_gather(b, b)

    def body(i, carry):
        for b in range(NGB):
            iloc = i * NGB + b
            tb = b % NTB
            gather_wait(b)

            @pl.when(iloc >= NTB)
            def _():
                out_wait(tb)

            transpose_unit(b, tb)
            out_start(ubase + iloc, tb)
            nxt = iloc + NGB

            @pl.when(nxt < UPW)
            def _():
                fire_gather(nxt, b)

        return carry

    lax.fori_loop(0, OUTER, body, 0)

    for tb in range(NTB):
        out_wait(tb)


NT_B = 1536
NT_MAIN = 999936
NT_BLOCKS = NT_MAIN // NT_B


def _pad_table_body(wt_ref, tail_ref, out_ref, vi0, vi1, vo0, vo1, *sems):
    vi = (vi0, vi1)
    vo = (vo0, vo1)
    si = sems[:2]
    so = sems[2:4]
    tsem = sems[4]

    def fire_in(t, b):
        pltpu.make_async_copy(wt_ref.at[:, pl.ds(t * NT_B, NT_B)], vi[b],
                              si[b]).start()

    def in_wait(b):
        pltpu.make_async_copy(wt_ref.at[:, pl.ds(0, NT_B)], vi[b],
                              si[b]).wait()

    def fire_out(t, b):
        pltpu.make_async_copy(vo[b], out_ref.at[pl.ds(t * NT_B, NT_B), :],
                              so[b]).start()

    def out_wait(b):
        pltpu.make_async_copy(vo[b], out_ref.at[pl.ds(0, NT_B), :],
                              so[b]).wait()

    def xp(b):
        t = vi[b][...].T
        vo[b][...] = jnp.concatenate([t, t], axis=1)

    pltpu.make_async_copy(tail_ref, out_ref.at[pl.ds(NT_MAIN, DIM), :],
                          tsem).start()

    for b in range(2):
        fire_in(b, b)

    def body(i, carry):
        for b in range(2):
            t = i * 2 + b
            in_wait(b)

            @pl.when(t >= 2)
            def _():
                out_wait(b)

            xp(b)
            fire_out(t, b)

            @pl.when(t + 2 < NT_BLOCKS)
            def _():
                fire_in(t + 2, b)

        return carry

    lax.fori_loop(0, NT_BLOCKS // 2, body, 0)

    in_wait(0)
    out_wait(0)
    xp(0)
    fire_out(NT_BLOCKS - 1, 0)

    for b in range(2):
        out_wait(b)
    pltpu.make_async_copy(tail_ref, out_ref.at[pl.ds(NT_MAIN, DIM), :],
                          tsem).wait()


_pad_table = pl.pallas_call(
    _pad_table_body,
    in_specs=[pl.BlockSpec(memory_space=pl.ANY),
              pl.BlockSpec(memory_space=pl.ANY)],
    out_specs=pl.BlockSpec(memory_space=pl.ANY),
    out_shape=jax.ShapeDtypeStruct((NEMB, 128), jnp.float32),
    scratch_shapes=(
        [pltpu.VMEM((DIM, NT_B), jnp.float32) for _ in range(2)]
        + [pltpu.VMEM((NT_B, 128), jnp.float32) for _ in range(2)]
        + [pltpu.SemaphoreType.DMA for _ in range(5)]
    ),
)


def kernel(token_ids, weight):
    idx = token_ids.T.reshape(NW, UPW, CHUNK).astype(jnp.int32)
    tail = jnp.pad(weight[NT_MAIN:], ((0, 0), (0, DIM)))
    table = _pad_table(weight.T, tail)
    out = _embedding_gather(table, idx)
    return out.transpose(2, 0, 1)


# speedup vs baseline: 1.0595x; 1.0595x over previous
import functools
import jax
import jax.numpy as jnp
from jax import lax
from jax.experimental import pallas as pl
from jax.experimental.pallas import tpu as pltpu
from jax.experimental.pallas import tpu_sc as plsc

NUM_TOKENS = 16384 * 50
DIM = 64
POS = 50
SEQ = 16384
NC, NS = 2, 16
NW = NC * NS
CHUNK = 128
UNITS = NUM_TOKENS // CHUNK
UPW = UNITS // NW
SBLK = SEQ // CHUNK
NBUF = 4
OUTER = UPW // NBUF
NUM_EMB_PAIRS = 500000


@functools.partial(
    pl.kernel,
    mesh=plsc.VectorSubcoreMesh(core_axis_name="c", subcore_axis_name="s"),
    out_type=jax.ShapeDtypeStruct((POS, DIM, SEQ), jnp.float32),
    scratch_types=(
        [pltpu.VMEM((UPW, CHUNK), jnp.int32)]
        + [pltpu.VMEM((CHUNK, 128), jnp.float32) for _ in range(NBUF)]
        + [pltpu.VMEM((DIM, CHUNK), jnp.float32) for _ in range(NBUF)]
        + [pltpu.VMEM((CHUNK,), jnp.int32) for _ in range(NBUF)]
        + [pltpu.SemaphoreType.DMA for _ in range(2 * NBUF)]
    ),
    compiler_params=pltpu.CompilerParams(use_tc_tiling_on_sc=True,
                                         needs_layout_passes=False),
)
def _embedding_gather(table_hbm, idx_hbm, out_hbm, idx_v, *scratch):
    gbuf = scratch[:NBUF]
    tbuf = scratch[NBUF:2 * NBUF]
    gidx = scratch[2 * NBUF:3 * NBUF]
    gsem = scratch[3 * NBUF:4 * NBUF]
    osem = scratch[4 * NBUF:]

    wid = lax.axis_index("s") * NC + lax.axis_index("c")
    ubase = wid * UPW

    pltpu.sync_copy(idx_hbm.at[wid], idx_v)

    lane = lax.iota(jnp.int32, 16)

    def fire_gather(jloc, b):
        for jg in range(CHUNK // 16):
            tid = idx_v[jloc, pl.ds(jg * 16, 16)]
            gidx[b][pl.ds(jg * 16, 16)] = jnp.where(
                tid >= NUM_EMB_PAIRS, tid - NUM_EMB_PAIRS, tid)
        pltpu.async_copy(table_hbm.at[gidx[b]], gbuf[b], gsem[b])

    def gather_wait(b):
        pltpu.make_async_copy(table_hbm.at[gidx[b]], gbuf[b], gsem[b]).wait()

    def transpose_unit(jloc, b):
        cols = []
        for jg in range(CHUNK // 16):
            tid = idx_v[jloc, pl.ds(jg * 16, 16)]
            cols.append(jnp.where(tid >= NUM_EMB_PAIRS, DIM, 0))
        rows = [jg * 16 + lane for jg in range(CHUNK // 16)]

        def dbody(db, carry):
            base = db * 16
            for s in range(16):
                t = (lane + s) & 15
                dvec = t + base
                for jg in range(CHUNK // 16):
                    v = plsc.load_gather(gbuf[b], [rows[jg], cols[jg] + dvec])
                    plsc.store_scatter(tbuf[b], [dvec, rows[jg]], v)
            return carry

        lax.fori_loop(0, DIM // 16, dbody, 0)

    def out_start(u, b):
        p = u // SBLK
        sb = u % SBLK
        pltpu.async_copy(tbuf[b], out_hbm.at[p, :, pl.ds(sb * CHUNK, CHUNK)],
                         osem[b])

    def out_wait(b):
        pltpu.make_async_copy(tbuf[b], out_hbm.at[0, :, pl.ds(0, CHUNK)],
                              osem[b]).wait()

    for b in range(NBUF):
        fire_gather(b, b)

    def body(i, carry):
        for b in range(NBUF):
            iloc = i * NBUF + b
            gather_wait(b)

            @pl.when(iloc >= NBUF)
            def _():
                out_wait(b)

            transpose_unit(iloc, b)
            out_start(ubase + iloc, b)
            nxt = iloc + NBUF

            @pl.when(nxt < UPW)
            def _():
                fire_gather(nxt, b)

        return carry

    lax.fori_loop(0, OUTER, body, 0)

    for b in range(NBUF):
        out_wait(b)


PACK_B = 4000


def _pack_kernel_body(top_ref, bot_ref, out_ref):
    out_ref[...] = jnp.concatenate([top_ref[...], bot_ref[...]], axis=1)


_pack_table = pl.pallas_call(
    _pack_kernel_body,
    grid=(NUM_EMB_PAIRS // PACK_B,),
    in_specs=[pl.BlockSpec((PACK_B, DIM), lambda i: (i, 0)),
              pl.BlockSpec((PACK_B, DIM),
                           lambda i: (i + NUM_EMB_PAIRS // PACK_B, 0))],
    out_specs=pl.BlockSpec((PACK_B, 128), lambda i: (i, 0)),
    out_shape=jax.ShapeDtypeStruct((NUM_EMB_PAIRS, 128), jnp.float32),
    compiler_params=pltpu.CompilerParams(
        dimension_semantics=("arbitrary",)),
)


def kernel(token_ids, weight):
    idx = token_ids.T.reshape(NW, UPW, CHUNK).astype(jnp.int32)
    table = _pack_table(weight, weight)
    out = _embedding_gather(table, idx)
    return out.transpose(2, 0, 1)
